# Initial kernel scaffold; baseline (speedup 1.0000x reference)
import jax
import jax.numpy as jnp
from jax.experimental import pallas as pl


def kernel(log_node_vt, v_pred, timestep, batch, log_alphas, log_1_min_alphas, log_cumprod_alphas, log_1_min_cumprod_alphas, uniform):
    raise NotImplementedError("write your pallas kernel here")


# speedup vs baseline: 7.6064x; 7.6064x over previous
import functools

import jax
import jax.numpy as jnp
from jax.experimental import pallas as pl
from jax.experimental.pallas import tpu as pltpu

_ROWS = 512


def _prep_body(ts_ref, tbl_ref, gt_ref):
    t = ts_ref[:, :]
    T = tbl_ref.shape[0]
    tm1 = jnp.maximum(t - 1, 0)
    tio = jax.lax.broadcasted_iota(jnp.int32, (1, T), 1)
    oh_tm1 = (tm1 == tio).astype(jnp.float32)
    oh_t = (t == tio).astype(jnp.float32)
    dn = (((1,), (0,)), ((), ()))
    cum = jax.lax.dot_general(oh_tm1, tbl_ref[:, 0:2], dn,
                              precision=jax.lax.Precision.HIGHEST,
                              preferred_element_type=jnp.float32)
    alp = jax.lax.dot_general(oh_t, tbl_ref[:, 2:4], dn,
                              precision=jax.lax.Precision.HIGHEST,
                              preferred_element_type=jnp.float32)
    is0 = t == 0
    inv_c = jnp.float32(1.0 / 128.0)
    a = jnp.where(is0, 1.0, jnp.exp(cum[:, 0:1]))
    b = jnp.where(is0, 0.0, jnp.exp(cum[:, 1:2]) * inv_c)
    c = jnp.exp(alp[:, 0:1])
    d = jnp.exp(alp[:, 1:2]) * inv_c
    gt_ref[:, :] = jnp.concatenate([a, b, c, d], axis=1)


def _main_body(gt_ref, batch_ref, lv_ref, vp_ref, u_ref,
               log_out_ref, idx_out_ref, oh_out_ref):
    R, C = lv_ref.shape
    B = gt_ref.shape[0]
    bidx = batch_ref[:, :]
    bio = jax.lax.broadcasted_iota(jnp.int32, (1, B), 1)
    oh = (bidx == bio).astype(jnp.float32)
    dn = (((1,), (0,)), ((), ()))
    coefs = jax.lax.dot_general(oh, gt_ref[:, :], dn,
                                precision=jax.lax.Precision.HIGHEST,
                                preferred_element_type=jnp.float32)
    a = coefs[:, 0:1]
    b = coefs[:, 1:2]
    c = coefs[:, 2:3]
    d = coefs[:, 3:4]

    vp = vp_ref[:, :]
    m = jnp.max(vp, axis=1, keepdims=True)
    e1 = jnp.exp(vp - m)
    s = jnp.sum(e1, axis=1, keepdims=True)
    x1 = a * e1 + b * s
    x2 = c * jnp.exp(lv_ref[:, :]) + d
    g = -jnp.log(u_ref[:, :] + 1e-30) + 1e-30
    w = (x1 * x2) / g

    wmax = jnp.max(w, axis=1, keepdims=True)
    cio = jax.lax.broadcasted_iota(jnp.int32, (R, C), 1)
    idx = jnp.min(jnp.where(w == wmax, cio, C), axis=1, keepdims=True)
    eq = cio == idx
    idx_out_ref[:, :] = idx
    oh_out_ref[:, :] = eq.astype(jnp.float32)
    log_out_ref[:, :] = jnp.where(eq, jnp.float32(0.0),
                                  jnp.log(jnp.float32(1e-30)))


@jax.jit
def kernel(log_node_vt, v_pred, timestep, batch, log_alphas, log_1_min_alphas,
           log_cumprod_alphas, log_1_min_cumprod_alphas, uniform):
    N, C = log_node_vt.shape
    B = timestep.shape[0]
    T = log_alphas.shape[0]
    R = _ROWS

    tbl = jnp.stack([log_cumprod_alphas, log_1_min_cumprod_alphas,
                     log_alphas, log_1_min_alphas], axis=1)
    ts2 = timestep.astype(jnp.int32).reshape(B, 1)
    batch2 = batch.astype(jnp.int32).reshape(N, 1)

    gt = pl.pallas_call(
        _prep_body,
        out_shape=jax.ShapeDtypeStruct((B, 4), jnp.float32),
    )(ts2, tbl)

    grid = (N // R,)
    log_out, idx_out, oh_out = pl.pallas_call(
        _main_body,
        grid=grid,
        in_specs=[
            pl.BlockSpec((B, 4), lambda i: (0, 0)),
            pl.BlockSpec((R, 1), lambda i: (i, 0)),
            pl.BlockSpec((R, C), lambda i: (i, 0)),
            pl.BlockSpec((R, C), lambda i: (i, 0)),
            pl.BlockSpec((R, C), lambda i: (i, 0)),
        ],
        out_specs=[
            pl.BlockSpec((R, C), lambda i: (i, 0)),
            pl.BlockSpec((R, 1), lambda i: (i, 0)),
            pl.BlockSpec((R, C), lambda i: (i, 0)),
        ],
        out_shape=[
            jax.ShapeDtypeStruct((N, C), jnp.float32),
            jax.ShapeDtypeStruct((N, 1), jnp.int32),
            jax.ShapeDtypeStruct((N, C), jnp.float32),
        ],
    )(gt, batch2, log_node_vt, v_pred, uniform)

    return (log_out, idx_out.reshape(N), oh_out)


# speedup vs baseline: 11.7348x; 1.5428x over previous
import functools

import jax
import jax.numpy as jnp
from jax import lax
from jax.experimental import pallas as pl
from jax.experimental.pallas import tpu as pltpu
from jax.experimental.pallas import tpu_sc as plsc

_ROWS = 512
_L = 16


def _prep_body(ts_ref, tbl_ref, gt_ref):
    t = ts_ref[:, :]
    T = tbl_ref.shape[1]
    B = ts_ref.shape[1]
    tm1 = jnp.maximum(t - 1, 0)
    tio = jax.lax.broadcasted_iota(jnp.int32, (T, 1), 0)
    oh_tm1 = (tio == tm1).astype(jnp.float32)
    oh_t = (tio == t).astype(jnp.float32)
    dn = (((1,), (0,)), ((), ()))
    cum = jax.lax.dot_general(tbl_ref[0:2, :], oh_tm1, dn,
                              precision=jax.lax.Precision.HIGHEST,
                              preferred_element_type=jnp.float32)
    alp = jax.lax.dot_general(tbl_ref[2:4, :], oh_t, dn,
                              precision=jax.lax.Precision.HIGHEST,
                              preferred_element_type=jnp.float32)
    is0 = t == 0
    inv_c = jnp.float32(1.0 / 128.0)
    a = jnp.where(is0, 1.0, jnp.exp(cum[0:1, :]))
    b = jnp.where(is0, 0.0, jnp.exp(cum[1:2, :]) * inv_c)
    c = jnp.exp(alp[0:1, :])
    d = jnp.exp(alp[1:2, :]) * inv_c
    pad = jnp.zeros((4, B), jnp.float32)
    gt_ref[:, :] = jnp.concatenate([a, b, c, d, pad], axis=0)


def _make_sc_gather(n_rows, n_graphs):
    info = plsc.get_sparse_core_info()
    nw = info.num_cores * info.num_subcores
    b_per_w = n_rows // nw
    n_iters = b_per_w // _L
    mesh = plsc.VectorSubcoreMesh(core_axis_name="c", subcore_axis_name="s")
    f32 = jnp.float32

    @functools.partial(
        pl.kernel,
        mesh=mesh,
        compiler_params=pltpu.CompilerParams(needs_layout_passes=False),
        out_type=[jax.ShapeDtypeStruct((n_rows,), f32) for _ in range(4)],
        scratch_types=[
            pltpu.VMEM((b_per_w,), jnp.int32),
            pltpu.VMEM((n_graphs,), f32),
            pltpu.VMEM((n_graphs,), f32),
            pltpu.VMEM((n_graphs,), f32),
            pltpu.VMEM((n_graphs,), f32),
            pltpu.VMEM((b_per_w,), f32),
            pltpu.VMEM((b_per_w,), f32),
            pltpu.VMEM((b_per_w,), f32),
            pltpu.VMEM((b_per_w,), f32),
        ],
    )
    def gather_k(gtab_hbm, batch_hbm, oa, ob, oc, od,
                 idx_v, gta, gtb, gtc, gtd, av, bv, cv, dv):
        wid = lax.axis_index("s") * info.num_cores + lax.axis_index("c")
        base = wid * b_per_w
        pltpu.sync_copy(gtab_hbm.at[0], gta)
        pltpu.sync_copy(gtab_hbm.at[1], gtb)
        pltpu.sync_copy(gtab_hbm.at[2], gtc)
        pltpu.sync_copy(gtab_hbm.at[3], gtd)
        pltpu.sync_copy(batch_hbm.at[pl.ds(base, b_per_w)], idx_v)

        def body(i):
            off = i * _L
            g = idx_v[pl.ds(off, _L)]
            av[pl.ds(off, _L)] = plsc.load_gather(gta, [g])
            bv[pl.ds(off, _L)] = plsc.load_gather(gtb, [g])
            cv[pl.ds(off, _L)] = plsc.load_gather(gtc, [g])
            dv[pl.ds(off, _L)] = plsc.load_gather(gtd, [g])

        pl.loop(0, n_iters)(body)
        pltpu.sync_copy(av, oa.at[pl.ds(base, b_per_w)])
        pltpu.sync_copy(bv, ob.at[pl.ds(base, b_per_w)])
        pltpu.sync_copy(cv, oc.at[pl.ds(base, b_per_w)])
        pltpu.sync_copy(dv, od.at[pl.ds(base, b_per_w)])

    return gather_k


def _main_body(a_ref, b_ref, c_ref, d_ref, lv_ref, vp_ref, u_ref,
               log_out_ref, idx_out_ref, oh_out_ref):
    R, C = lv_ref.shape
    a = a_ref[:, :]
    b = b_ref[:, :]
    c = c_ref[:, :]
    d = d_ref[:, :]

    e1 = jnp.exp(vp_ref[:, :])
    s = jnp.sum(e1, axis=1, keepdims=True)
    x1 = a * e1 + b * s
    x2 = c * jnp.exp(lv_ref[:, :]) + d
    g = -jnp.log(u_ref[:, :] + 1e-30) + 1e-30
    w = (x1 * x2) / g

    wmax = jnp.max(w, axis=1, keepdims=True)
    cio = jax.lax.broadcasted_iota(jnp.int32, (R, C), 1)
    idx = jnp.min(jnp.where(w == wmax, cio, C), axis=1, keepdims=True)
    eq = cio == idx
    idx_out_ref[:, :] = idx
    oh_out_ref[:, :] = eq.astype(jnp.float32)
    log_out_ref[:, :] = jnp.where(eq, jnp.float32(0.0),
                                  jnp.log(jnp.float32(1e-30)))


@jax.jit
def kernel(log_node_vt, v_pred, timestep, batch, log_alphas, log_1_min_alphas,
           log_cumprod_alphas, log_1_min_cumprod_alphas, uniform):
    N, C = log_node_vt.shape
    B = timestep.shape[0]
    R = _ROWS

    tbl = jnp.stack([log_cumprod_alphas, log_1_min_cumprod_alphas,
                     log_alphas, log_1_min_alphas], axis=0)
    ts2 = timestep.astype(jnp.int32).reshape(1, B)
    batch1 = batch.astype(jnp.int32)

    gtab = pl.pallas_call(
        _prep_body,
        out_shape=jax.ShapeDtypeStruct((8, B), jnp.float32),
    )(ts2, tbl)

    ca, cb, cc, cd = _make_sc_gather(N, B)(gtab, batch1)
    ca, cb, cc, cd = (x.reshape(N, 1) for x in (ca, cb, cc, cd))

    grid = (N // R,)
    log_out, idx_out, oh_out = pl.pallas_call(
        _main_body,
        grid=grid,
        in_specs=[
            pl.BlockSpec((R, 1), lambda i: (i, 0)),
            pl.BlockSpec((R, 1), lambda i: (i, 0)),
            pl.BlockSpec((R, 1), lambda i: (i, 0)),
            pl.BlockSpec((R, 1), lambda i: (i, 0)),
            pl.BlockSpec((R, C), lambda i: (i, 0)),
            pl.BlockSpec((R, C), lambda i: (i, 0)),
            pl.BlockSpec((R, C), lambda i: (i, 0)),
        ],
        out_specs=[
            pl.BlockSpec((R, C), lambda i: (i, 0)),
            pl.BlockSpec((R, 1), lambda i: (i, 0)),
            pl.BlockSpec((R, C), lambda i: (i, 0)),
        ],
        out_shape=[
            jax.ShapeDtypeStruct((N, C), jnp.float32),
            jax.ShapeDtypeStruct((N, 1), jnp.int32),
            jax.ShapeDtypeStruct((N, C), jnp.float32),
        ],
    )(ca, cb, cc, cd, log_node_vt, v_pred, uniform)

    return (log_out, idx_out.reshape(N), oh_out)


# speedup vs baseline: 22.8050x; 1.9434x over previous
import functools

import jax
import jax.numpy as jnp
from jax import lax
from jax.experimental import pallas as pl
from jax.experimental.pallas import tpu as pltpu
from jax.experimental.pallas import tpu_sc as plsc

_ROWS = 1024
_L = 16


def _prep_body(ts_ref, tbl_ref, gt_ref):
    t = ts_ref[:, :]
    T = tbl_ref.shape[1]
    B = ts_ref.shape[1]
    tm1 = jnp.maximum(t - 1, 0)
    tio = jax.lax.broadcasted_iota(jnp.int32, (T, 1), 0)
    oh_tm1 = (tio == tm1).astype(jnp.float32)
    oh_t = (tio == t).astype(jnp.float32)
    dn = (((1,), (0,)), ((), ()))
    cum = jax.lax.dot_general(tbl_ref[0:2, :], oh_tm1, dn,
                              precision=jax.lax.Precision.HIGHEST,
                              preferred_element_type=jnp.float32)
    alp = jax.lax.dot_general(tbl_ref[2:4, :], oh_t, dn,
                              precision=jax.lax.Precision.HIGHEST,
                              preferred_element_type=jnp.float32)
    is0 = t == 0
    inv_c = jnp.float32(1.0 / 128.0)
    a = jnp.where(is0, 1.0, jnp.exp(cum[0:1, :]))
    b = jnp.where(is0, 0.0, jnp.exp(cum[1:2, :]) * inv_c)
    c = jnp.exp(alp[0:1, :])
    d = jnp.exp(alp[1:2, :]) * inv_c
    pad = jnp.zeros((4, B), jnp.float32)
    gt_ref[:, :] = jnp.concatenate([a, b, c, d, pad], axis=0)


def _make_sc_gather(n_rows, n_graphs, rows_per_step):
    info = plsc.get_sparse_core_info()
    nw = info.num_cores * info.num_subcores
    b_per_w = n_rows // nw
    n_iters = b_per_w // _L
    chunks_per_step = rows_per_step // _L
    out_rows = 4 * n_rows // 128
    w_rows = out_rows // nw
    mesh = plsc.VectorSubcoreMesh(core_axis_name="c", subcore_axis_name="s")
    f32 = jnp.float32

    @functools.partial(
        pl.kernel,
        mesh=mesh,
        compiler_params=pltpu.CompilerParams(needs_layout_passes=False),
        out_type=jax.ShapeDtypeStruct((out_rows, 128), f32),
        scratch_types=[
            pltpu.VMEM((b_per_w,), jnp.int32),
            pltpu.VMEM((n_graphs,), f32),
            pltpu.VMEM((n_graphs,), f32),
            pltpu.VMEM((n_graphs,), f32),
            pltpu.VMEM((n_graphs,), f32),
            pltpu.VMEM((w_rows, 128), f32),
        ],
    )
    def gather_k(gtab_hbm, batch_hbm, out_hbm,
                 idx_v, gta, gtb, gtc, gtd, pv):
        wid = lax.axis_index("s") * info.num_cores + lax.axis_index("c")
        base = wid * b_per_w
        pltpu.sync_copy(gtab_hbm.at[0], gta)
        pltpu.sync_copy(gtab_hbm.at[1], gtb)
        pltpu.sync_copy(gtab_hbm.at[2], gtc)
        pltpu.sync_copy(gtab_hbm.at[3], gtd)
        pltpu.sync_copy(batch_hbm.at[pl.ds(base, b_per_w)], idx_v)

        def body(i):
            g = idx_v[pl.ds(i * _L, _L)]
            j = i // chunks_per_step
            k0 = (i % chunks_per_step) * _L
            o = j * (4 * rows_per_step) + k0
            pv[(o // 128), pl.ds(o % 128, _L)] = plsc.load_gather(gta, [g])
            o = o + rows_per_step
            pv[(o // 128), pl.ds(o % 128, _L)] = plsc.load_gather(gtb, [g])
            o = o + rows_per_step
            pv[(o // 128), pl.ds(o % 128, _L)] = plsc.load_gather(gtc, [g])
            o = o + rows_per_step
            pv[(o // 128), pl.ds(o % 128, _L)] = plsc.load_gather(gtd, [g])

        pl.loop(0, n_iters)(body)
        pltpu.sync_copy(pv, out_hbm.at[pl.ds(wid * w_rows, w_rows), :])

    return gather_k


def _main_body(a_ref, b_ref, c_ref, d_ref, lv_ref, vp_ref, u_ref,
               log_out_ref, idx_out_ref, oh_out_ref):
    R, C = lv_ref.shape
    nsub = R // 128
    stack4 = jnp.concatenate(
        [a_ref[:, :], b_ref[:, :], c_ref[:, :], d_ref[:, :]], axis=0)
    rio = jax.lax.broadcasted_iota(jnp.int32, (128, 128), 0)
    lio = jax.lax.broadcasted_iota(jnp.int32, (128, 128), 1)
    ident = (rio == lio).astype(jnp.float32)
    dnt = (((1,), (1,)), ((), ()))
    tc = jax.lax.dot_general(ident, stack4, dnt,
                             precision=jax.lax.Precision.HIGHEST,
                             preferred_element_type=jnp.float32)

    a = jnp.concatenate([tc[:, j:j + 1] for j in range(nsub)], axis=0)
    b = jnp.concatenate([tc[:, nsub + j:nsub + j + 1]
                         for j in range(nsub)], axis=0)
    c = jnp.concatenate([tc[:, 2 * nsub + j:2 * nsub + j + 1]
                         for j in range(nsub)], axis=0)
    d = jnp.concatenate([tc[:, 3 * nsub + j:3 * nsub + j + 1]
                         for j in range(nsub)], axis=0)

    e1 = jnp.exp(vp_ref[:, :])
    s = jnp.sum(e1, axis=1, keepdims=True)
    x1 = a * e1 + b * s
    x2 = c * jnp.exp(lv_ref[:, :]) + d
    g = -jnp.log(u_ref[:, :] + 1e-30) + 1e-30
    w = (x1 * x2) / g

    wmax = jnp.max(w, axis=1, keepdims=True)
    cio = jax.lax.broadcasted_iota(jnp.int32, (R, C), 1)
    idx = jnp.min(jnp.where(w == wmax, cio, C), axis=1, keepdims=True)
    eq = cio == idx
    oh_out_ref[:, :] = eq.astype(jnp.float32)
    log_out_ref[:, :] = jnp.where(eq, jnp.float32(0.0),
                                  jnp.log(jnp.float32(1e-30)))
    icols = jnp.concatenate(
        [idx[j * 128:(j + 1) * 128, :] for j in range(nsub)],
        axis=1).astype(jnp.float32)
    dnr = (((0,), (0,)), ((), ()))
    irows = jax.lax.dot_general(icols, ident, dnr,
                                preferred_element_type=jnp.float32)
    idx_out_ref[:, :] = irows.astype(jnp.int32)


@jax.jit
def kernel(log_node_vt, v_pred, timestep, batch, log_alphas, log_1_min_alphas,
           log_cumprod_alphas, log_1_min_cumprod_alphas, uniform):
    N, C = log_node_vt.shape
    B = timestep.shape[0]
    R = _ROWS

    tbl = jnp.stack([log_cumprod_alphas, log_1_min_cumprod_alphas,
                     log_alphas, log_1_min_alphas], axis=0)
    ts2 = timestep.astype(jnp.int32).reshape(1, B)
    batch1 = batch.astype(jnp.int32)

    gtab = pl.pallas_call(
        _prep_body,
        out_shape=jax.ShapeDtypeStruct((8, B), jnp.float32),
    )(ts2, tbl)

    coefs = _make_sc_gather(N, B, R)(gtab, batch1)

    rb = R // 128
    grid = (N // R,)
    log_out, idx_out, oh_out = pl.pallas_call(
        _main_body,
        grid=grid,
        in_specs=[
            pl.BlockSpec((rb, 128), lambda i: (4 * i + 0, 0)),
            pl.BlockSpec((rb, 128), lambda i: (4 * i + 1, 0)),
            pl.BlockSpec((rb, 128), lambda i: (4 * i + 2, 0)),
            pl.BlockSpec((rb, 128), lambda i: (4 * i + 3, 0)),
            pl.BlockSpec((R, C), lambda i: (i, 0)),
            pl.BlockSpec((R, C), lambda i: (i, 0)),
            pl.BlockSpec((R, C), lambda i: (i, 0)),
        ],
        out_specs=[
            pl.BlockSpec((R, C), lambda i: (i, 0)),
            pl.BlockSpec((R // 128, 128), lambda i: (i, 0)),
            pl.BlockSpec((R, C), lambda i: (i, 0)),
        ],
        out_shape=[
            jax.ShapeDtypeStruct((N, C), jnp.float32),
            jax.ShapeDtypeStruct((N // 128, 128), jnp.int32),
            jax.ShapeDtypeStruct((N, C), jnp.float32),
        ],
    )(coefs, coefs, coefs, coefs, log_node_vt, v_pred, uniform)

    return (log_out, idx_out.reshape(N), oh_out)


# speedup vs baseline: 26.0982x; 1.1444x over previous
import functools

import jax
import jax.numpy as jnp
from jax import lax
from jax.experimental import pallas as pl
from jax.experimental.pallas import tpu as pltpu
from jax.experimental.pallas import tpu_sc as plsc

_ROWS = 1024
_L = 16


def _prep_body(ts_ref, tbl_ref, gt_ref):
    t = ts_ref[:, :]
    T = tbl_ref.shape[1]
    B = ts_ref.shape[1]
    tm1 = jnp.maximum(t - 1, 0)
    tio = jax.lax.broadcasted_iota(jnp.int32, (T, 1), 0)
    oh_tm1 = (tio == tm1).astype(jnp.float32)
    oh_t = (tio == t).astype(jnp.float32)
    dn = (((1,), (0,)), ((), ()))
    cum = jax.lax.dot_general(tbl_ref[0:2, :], oh_tm1, dn,
                              precision=jax.lax.Precision.HIGHEST,
                              preferred_element_type=jnp.float32)
    alp = jax.lax.dot_general(tbl_ref[2:4, :], oh_t, dn,
                              precision=jax.lax.Precision.HIGHEST,
                              preferred_element_type=jnp.float32)
    is0 = t == 0
    inv_c = jnp.float32(1.0 / 128.0)
    ba = jnp.where(is0, 0.0,
                   jnp.exp(cum[1:2, :] - cum[0:1, :]) * inv_c)
    rho = jnp.exp(alp[1:2, :] - alp[0:1, :]) * inv_c
    pad = jnp.zeros((6, B), jnp.float32)
    gt_ref[:, :] = jnp.concatenate([ba, rho, pad], axis=0)


def _make_sc_gather(n_rows, n_graphs, rows_per_step):
    info = plsc.get_sparse_core_info()
    nw = info.num_cores * info.num_subcores
    b_per_w = n_rows // nw
    n_iters = b_per_w // _L
    chunks_per_step = rows_per_step // _L
    out_rows = 2 * n_rows // 128
    w_rows = out_rows // nw
    mesh = plsc.VectorSubcoreMesh(core_axis_name="c", subcore_axis_name="s")
    f32 = jnp.float32

    @functools.partial(
        pl.kernel,
        mesh=mesh,
        compiler_params=pltpu.CompilerParams(needs_layout_passes=False),
        out_type=jax.ShapeDtypeStruct((out_rows, 128), f32),
        scratch_types=[
            pltpu.VMEM((b_per_w,), jnp.int32),
            pltpu.VMEM((n_graphs,), f32),
            pltpu.VMEM((n_graphs,), f32),
            pltpu.VMEM((w_rows, 128), f32),
        ],
    )
    def gather_k(gtab_hbm, batch_hbm, out_hbm, idx_v, gta, gtb, pv):
        wid = lax.axis_index("s") * info.num_cores + lax.axis_index("c")
        base = wid * b_per_w
        pltpu.sync_copy(gtab_hbm.at[0], gta)
        pltpu.sync_copy(gtab_hbm.at[1], gtb)
        pltpu.sync_copy(batch_hbm.at[pl.ds(base, b_per_w)], idx_v)

        def body(i):
            g = idx_v[pl.ds(i * _L, _L)]
            j = i // chunks_per_step
            k0 = (i % chunks_per_step) * _L
            o = j * (2 * rows_per_step) + k0
            pv[(o // 128), pl.ds(o % 128, _L)] = plsc.load_gather(gta, [g])
            o = o + rows_per_step
            pv[(o // 128), pl.ds(o % 128, _L)] = plsc.load_gather(gtb, [g])

        pl.loop(0, n_iters)(body)
        pltpu.sync_copy(pv, out_hbm.at[pl.ds(wid * w_rows, w_rows), :])

    return gather_k


def _main_body(a_ref, b_ref, lv_ref, vp_ref, u_ref,
               log_out_ref, idx_out_ref, oh_out_ref):
    R, C = lv_ref.shape
    nsub = R // 128
    stack2 = jnp.concatenate([a_ref[:, :], b_ref[:, :]], axis=0)
    rio = jax.lax.broadcasted_iota(jnp.int32, (128, 128), 0)
    lio = jax.lax.broadcasted_iota(jnp.int32, (128, 128), 1)
    ident = (rio == lio).astype(jnp.float32)
    dnt = (((1,), (1,)), ((), ()))
    tc = jax.lax.dot_general(ident, stack2, dnt,
                             precision=jax.lax.Precision.HIGHEST,
                             preferred_element_type=jnp.float32)

    ba = jnp.concatenate([tc[:, j:j + 1] for j in range(nsub)], axis=0)
    rho = jnp.concatenate([tc[:, nsub + j:nsub + j + 1]
                           for j in range(nsub)], axis=0)

    e1 = jnp.exp(vp_ref[:, :])
    s = jnp.sum(e1, axis=1, keepdims=True)
    tau = ba * s
    x1 = e1 + tau
    x2 = jnp.exp(lv_ref[:, :]) + rho
    g = -jnp.log(u_ref[:, :] + 1e-30)
    w = (x1 * x2) / g

    wmax = jnp.max(w, axis=1, keepdims=True)
    cio = jax.lax.broadcasted_iota(jnp.int32, (R, C), 1)
    eq = w == wmax
    idx = jnp.min(jnp.where(eq, cio, C), axis=1, keepdims=True)
    oh_out_ref[:, :] = eq.astype(jnp.float32)
    log_out_ref[:, :] = jnp.where(eq, jnp.float32(0.0),
                                  jnp.log(jnp.float32(1e-30)))
    icols = jnp.concatenate(
        [idx[j * 128:(j + 1) * 128, :] for j in range(nsub)],
        axis=1).astype(jnp.float32)
    dnr = (((0,), (0,)), ((), ()))
    irows = jax.lax.dot_general(icols, ident, dnr,
                                preferred_element_type=jnp.float32)
    idx_out_ref[:, :] = irows.astype(jnp.int32)


@jax.jit
def kernel(log_node_vt, v_pred, timestep, batch, log_alphas, log_1_min_alphas,
           log_cumprod_alphas, log_1_min_cumprod_alphas, uniform):
    N, C = log_node_vt.shape
    B = timestep.shape[0]
    R = _ROWS

    tbl = jnp.stack([log_cumprod_alphas, log_1_min_cumprod_alphas,
                     log_alphas, log_1_min_alphas], axis=0)
    ts2 = timestep.astype(jnp.int32).reshape(1, B)
    batch1 = batch.astype(jnp.int32)

    gtab = pl.pallas_call(
        _prep_body,
        out_shape=jax.ShapeDtypeStruct((8, B), jnp.float32),
    )(ts2, tbl)

    coefs = _make_sc_gather(N, B, R)(gtab, batch1)

    rb = R // 128
    grid = (N // R,)
    log_out, idx_out, oh_out = pl.pallas_call(
        _main_body,
        grid=grid,
        in_specs=[
            pl.BlockSpec((rb, 128), lambda i: (2 * i + 0, 0)),
            pl.BlockSpec((rb, 128), lambda i: (2 * i + 1, 0)),
            pl.BlockSpec((R, C), lambda i: (i, 0)),
            pl.BlockSpec((R, C), lambda i: (i, 0)),
            pl.BlockSpec((R, C), lambda i: (i, 0)),
        ],
        out_specs=[
            pl.BlockSpec((R, C), lambda i: (i, 0)),
            pl.BlockSpec((R // 128, 128), lambda i: (i, 0)),
            pl.BlockSpec((R, C), lambda i: (i, 0)),
        ],
        out_shape=[
            jax.ShapeDtypeStruct((N, C), jnp.float32),
            jax.ShapeDtypeStruct((N // 128, 128), jnp.int32),
            jax.ShapeDtypeStruct((N, C), jnp.float32),
        ],
    )(coefs, coefs, log_node_vt, v_pred, uniform)

    return (log_out, idx_out.reshape(N), oh_out)


# speedup vs baseline: 26.9593x; 1.0330x over previous
import functools

import jax
import jax.numpy as jnp
from jax import lax
from jax.experimental import pallas as pl
from jax.experimental.pallas import tpu as pltpu
from jax.experimental.pallas import tpu_sc as plsc

_ROWS = 1024
_L = 16


def _prep_body(ts_ref, tbl_ref, gt_ref):
    t = ts_ref[:, :]
    T = tbl_ref.shape[1]
    B = ts_ref.shape[1]
    tm1 = jnp.maximum(t - 1, 0)
    tio = jax.lax.broadcasted_iota(jnp.int32, (T, 1), 0)
    oh_tm1 = (tio == tm1).astype(jnp.float32)
    oh_t = (tio == t).astype(jnp.float32)
    dn = (((1,), (0,)), ((), ()))
    cum = jax.lax.dot_general(tbl_ref[0:2, :], oh_tm1, dn,
                              precision=jax.lax.Precision.HIGHEST,
                              preferred_element_type=jnp.float32)
    alp = jax.lax.dot_general(tbl_ref[2:4, :], oh_t, dn,
                              precision=jax.lax.Precision.HIGHEST,
                              preferred_element_type=jnp.float32)
    is0 = t == 0
    inv_c = jnp.float32(1.0 / 128.0)
    ba = jnp.where(is0, 0.0,
                   jnp.exp(cum[1:2, :] - cum[0:1, :]) * inv_c)
    rho = jnp.exp(alp[1:2, :] - alp[0:1, :]) * inv_c
    pad = jnp.zeros((6, B), jnp.float32)
    gt_ref[:, :] = jnp.concatenate([ba, rho, pad], axis=0)


def _make_sc_gather(n_rows, n_graphs, rows_per_step):
    info = plsc.get_sparse_core_info()
    nw = info.num_cores * info.num_subcores
    b_per_w = n_rows // nw
    n_iters = b_per_w // _L
    chunks_per_step = rows_per_step // _L
    out_rows = 2 * n_rows // 128
    w_rows = out_rows // nw
    mesh = plsc.VectorSubcoreMesh(core_axis_name="c", subcore_axis_name="s")
    f32 = jnp.float32

    @functools.partial(
        pl.kernel,
        mesh=mesh,
        compiler_params=pltpu.CompilerParams(needs_layout_passes=False),
        out_type=jax.ShapeDtypeStruct((out_rows, 128), f32),
        scratch_types=[
            pltpu.VMEM((b_per_w,), jnp.int32),
            pltpu.VMEM((n_graphs,), f32),
            pltpu.VMEM((n_graphs,), f32),
            pltpu.VMEM((w_rows, 128), f32),
        ],
    )
    def gather_k(gtab_hbm, batch_hbm, out_hbm, idx_v, gta, gtb, pv):
        wid = lax.axis_index("s") * info.num_cores + lax.axis_index("c")
        base = wid * b_per_w
        pltpu.sync_copy(gtab_hbm.at[0], gta)
        pltpu.sync_copy(gtab_hbm.at[1], gtb)
        pltpu.sync_copy(batch_hbm.at[pl.ds(base, b_per_w)], idx_v)

        def body(i):
            g = idx_v[pl.ds(i * _L, _L)]
            j = i // chunks_per_step
            k0 = (i % chunks_per_step) * _L
            o = j * (2 * rows_per_step) + k0
            pv[(o // 128), pl.ds(o % 128, _L)] = plsc.load_gather(gta, [g])
            o = o + rows_per_step
            pv[(o // 128), pl.ds(o % 128, _L)] = plsc.load_gather(gtb, [g])

        pl.loop(0, n_iters)(body)
        pltpu.sync_copy(pv, out_hbm.at[pl.ds(wid * w_rows, w_rows), :])

    return gather_k


def _main_body(a_ref, b_ref, lv_ref, vp_ref, u_ref,
               log_out_ref, idx_out_ref, oh_out_ref):
    R, C = lv_ref.shape
    nsub = R // 128
    stack2 = jnp.concatenate([a_ref[:, :], b_ref[:, :]], axis=0)
    rio = jax.lax.broadcasted_iota(jnp.int32, (128, 128), 0)
    lio = jax.lax.broadcasted_iota(jnp.int32, (128, 128), 1)
    ident = (rio == lio).astype(jnp.float32)
    dnt = (((1,), (1,)), ((), ()))
    tc = jax.lax.dot_general(ident, stack2, dnt,
                             precision=jax.lax.Precision.HIGHEST,
                             preferred_element_type=jnp.float32)

    ba = jnp.concatenate([tc[:, j:j + 1] for j in range(nsub)], axis=0)
    rho = jnp.concatenate([tc[:, nsub + j:nsub + j + 1]
                           for j in range(nsub)], axis=0)

    e1 = jnp.exp(vp_ref[:, :])
    s = jnp.sum(e1, axis=1, keepdims=True)
    tau = ba * s
    x1 = e1 + tau
    x2 = jnp.exp(lv_ref[:, :]) + rho
    g = -jnp.log(u_ref[:, :] + 1e-30)
    w = (x1 * x2) / g

    wmax = jnp.max(w, axis=1, keepdims=True)
    cio = jax.lax.broadcasted_iota(jnp.int32, (R, C), 1).astype(jnp.float32)
    eq = w == wmax
    idx = jnp.min(jnp.where(eq, cio, jnp.float32(C)), axis=1, keepdims=True)
    oh_out_ref[:, :] = eq.astype(jnp.float32)
    log_out_ref[:, :] = jnp.where(eq, jnp.float32(0.0),
                                  jnp.log(jnp.float32(1e-30)))
    icols = jnp.concatenate(
        [idx[j * 128:(j + 1) * 128, :] for j in range(nsub)],
        axis=1)
    dnr = (((0,), (0,)), ((), ()))
    irows = jax.lax.dot_general(icols, ident, dnr,
                                preferred_element_type=jnp.float32)
    idx_out_ref[:, :] = irows.astype(jnp.int32)


@jax.jit
def kernel(log_node_vt, v_pred, timestep, batch, log_alphas, log_1_min_alphas,
           log_cumprod_alphas, log_1_min_cumprod_alphas, uniform):
    N, C = log_node_vt.shape
    B = timestep.shape[0]
    R = _ROWS

    tbl = jnp.stack([log_cumprod_alphas, log_1_min_cumprod_alphas,
                     log_alphas, log_1_min_alphas], axis=0)
    ts2 = timestep.astype(jnp.int32).reshape(1, B)
    batch1 = batch.astype(jnp.int32)

    gtab = pl.pallas_call(
        _prep_body,
        out_shape=jax.ShapeDtypeStruct((8, B), jnp.float32),
    )(ts2, tbl)

    coefs = _make_sc_gather(N, B, R)(gtab, batch1)

    rb = R // 128
    grid = (N // R,)
    log_out, idx_out, oh_out = pl.pallas_call(
        _main_body,
        grid=grid,
        in_specs=[
            pl.BlockSpec((rb, 128), lambda i: (2 * i + 0, 0)),
            pl.BlockSpec((rb, 128), lambda i: (2 * i + 1, 0)),
            pl.BlockSpec((R, C), lambda i: (i, 0)),
            pl.BlockSpec((R, C), lambda i: (i, 0)),
            pl.BlockSpec((R, C), lambda i: (i, 0)),
        ],
        out_specs=[
            pl.BlockSpec((R, C), lambda i: (i, 0)),
            pl.BlockSpec((R // 128, 128), lambda i: (i, 0)),
            pl.BlockSpec((R, C), lambda i: (i, 0)),
        ],
        out_shape=[
            jax.ShapeDtypeStruct((N, C), jnp.float32),
            jax.ShapeDtypeStruct((N // 128, 128), jnp.int32),
            jax.ShapeDtypeStruct((N, C), jnp.float32),
        ],
    )(coefs, coefs, log_node_vt, v_pred, uniform)

    return (log_out, idx_out.reshape(N), oh_out)


# speedup vs baseline: 33.5073x; 1.2429x over previous
import functools

import jax
import jax.numpy as jnp
from jax import lax
from jax.experimental import pallas as pl
from jax.experimental.pallas import tpu as pltpu
from jax.experimental.pallas import tpu_sc as plsc

_ROWS = 2048
_L = 16


def _prep_body(ts_ref, tbl_ref, gt_ref):
    t = ts_ref[:, :]
    T = tbl_ref.shape[1]
    B = ts_ref.shape[1]
    tm1 = jnp.maximum(t - 1, 0)
    tio = jax.lax.broadcasted_iota(jnp.int32, (T, 1), 0)
    oh_tm1 = (tio == tm1).astype(jnp.float32)
    oh_t = (tio == t).astype(jnp.float32)
    dn = (((1,), (0,)), ((), ()))
    cum = jax.lax.dot_general(tbl_ref[0:2, :], oh_tm1, dn,
                              precision=jax.lax.Precision.HIGHEST,
                              preferred_element_type=jnp.float32)
    alp = jax.lax.dot_general(tbl_ref[2:4, :], oh_t, dn,
                              precision=jax.lax.Precision.HIGHEST,
                              preferred_element_type=jnp.float32)
    is0 = t == 0
    inv_c = jnp.float32(1.0 / 128.0)
    ba = jnp.where(is0, 0.0,
                   jnp.exp(cum[1:2, :] - cum[0:1, :]) * inv_c)
    rho = jnp.exp(alp[1:2, :] - alp[0:1, :]) * inv_c
    pad = jnp.zeros((6, B), jnp.float32)
    gt_ref[:, :] = jnp.concatenate([ba, rho, pad], axis=0)


def _make_sc_gather(n_rows, n_graphs, rows_per_step):
    info = plsc.get_sparse_core_info()
    nw = info.num_cores * info.num_subcores
    b_per_w = n_rows // nw
    n_iters = b_per_w // _L
    chunks_per_step = rows_per_step // _L
    out_rows = 2 * n_rows // 128
    w_rows = out_rows // nw
    mesh = plsc.VectorSubcoreMesh(core_axis_name="c", subcore_axis_name="s")
    f32 = jnp.float32

    @functools.partial(
        pl.kernel,
        mesh=mesh,
        compiler_params=pltpu.CompilerParams(needs_layout_passes=False),
        out_type=jax.ShapeDtypeStruct((out_rows, 128), f32),
        scratch_types=[
            pltpu.VMEM((b_per_w,), jnp.int32),
            pltpu.VMEM((n_graphs,), f32),
            pltpu.VMEM((n_graphs,), f32),
            pltpu.VMEM((w_rows, 128), f32),
        ],
    )
    def gather_k(gtab_hbm, batch_hbm, out_hbm, idx_v, gta, gtb, pv):
        wid = lax.axis_index("s") * info.num_cores + lax.axis_index("c")
        base = wid * b_per_w
        pltpu.sync_copy(gtab_hbm.at[0], gta)
        pltpu.sync_copy(gtab_hbm.at[1], gtb)
        pltpu.sync_copy(batch_hbm.at[pl.ds(base, b_per_w)], idx_v)

        def body(i):
            g = idx_v[pl.ds(i * _L, _L)]
            j = i // chunks_per_step
            k0 = (i % chunks_per_step) * _L
            o = j * (2 * rows_per_step) + k0
            pv[(o // 128), pl.ds(o % 128, _L)] = plsc.load_gather(gta, [g])
            o = o + rows_per_step
            pv[(o // 128), pl.ds(o % 128, _L)] = plsc.load_gather(gtb, [g])

        pl.loop(0, n_iters)(body)
        pltpu.sync_copy(pv, out_hbm.at[pl.ds(wid * w_rows, w_rows), :])

    return gather_k


def _main_body(a_ref, b_ref, lv_ref, vp_ref, u_ref,
               log_out_ref, idx_out_ref, oh_out_ref):
    R, C = lv_ref.shape
    nsub = R // 128
    stack2 = jnp.concatenate([a_ref[:, :], b_ref[:, :]], axis=0)
    rio = jax.lax.broadcasted_iota(jnp.int32, (128, 128), 0)
    lio = jax.lax.broadcasted_iota(jnp.int32, (128, 128), 1)
    ident = (rio == lio).astype(jnp.float32)
    dnt = (((1,), (1,)), ((), ()))
    tc = jax.lax.dot_general(ident, stack2, dnt,
                             precision=jax.lax.Precision.HIGHEST,
                             preferred_element_type=jnp.float32)

    ba = jnp.concatenate([tc[:, j:j + 1] for j in range(nsub)], axis=0)
    rho = jnp.concatenate([tc[:, nsub + j:nsub + j + 1]
                           for j in range(nsub)], axis=0)

    e1 = jnp.exp(vp_ref[:, :])
    s = jnp.sum(e1, axis=1, keepdims=True)
    tau = ba * s
    x1 = e1 + tau
    x2 = jnp.exp(lv_ref[:, :]) + rho
    g = -jnp.log(u_ref[:, :] + 1e-30)
    w = (x1 * x2) / g

    wmax = jnp.max(w, axis=1, keepdims=True)
    cio = jax.lax.broadcasted_iota(jnp.int32, (R, C), 1).astype(jnp.float32)
    eq = w == wmax
    idx = jnp.min(jnp.where(eq, cio, jnp.float32(C)), axis=1, keepdims=True)
    oh_out_ref[:, :] = eq.astype(jnp.float32)
    log_out_ref[:, :] = jnp.where(eq, jnp.float32(0.0),
                                  jnp.log(jnp.float32(1e-30)))
    icols = jnp.concatenate(
        [idx[j * 128:(j + 1) * 128, :] for j in range(nsub)],
        axis=1)
    dnr = (((0,), (0,)), ((), ()))
    irows = jax.lax.dot_general(icols, ident, dnr,
                                preferred_element_type=jnp.float32)
    idx_out_ref[:, :] = irows.astype(jnp.int32)


@jax.jit
def kernel(log_node_vt, v_pred, timestep, batch, log_alphas, log_1_min_alphas,
           log_cumprod_alphas, log_1_min_cumprod_alphas, uniform):
    N, C = log_node_vt.shape
    B = timestep.shape[0]
    R = _ROWS

    tbl = jnp.stack([log_cumprod_alphas, log_1_min_cumprod_alphas,
                     log_alphas, log_1_min_alphas], axis=0)
    ts2 = timestep.astype(jnp.int32).reshape(1, B)
    batch1 = batch.astype(jnp.int32)

    gtab = pl.pallas_call(
        _prep_body,
        out_shape=jax.ShapeDtypeStruct((8, B), jnp.float32),
    )(ts2, tbl)

    coefs = _make_sc_gather(N, B, R)(gtab, batch1)

    rb = R // 128
    grid = (N // R,)
    log_out, idx_out, oh_out = pl.pallas_call(
        _main_body,
        grid=grid,
        in_specs=[
            pl.BlockSpec((rb, 128), lambda i: (2 * i + 0, 0)),
            pl.BlockSpec((rb, 128), lambda i: (2 * i + 1, 0)),
            pl.BlockSpec((R, C), lambda i: (i, 0)),
            pl.BlockSpec((R, C), lambda i: (i, 0)),
            pl.BlockSpec((R, C), lambda i: (i, 0)),
        ],
        out_specs=[
            pl.BlockSpec((R, C), lambda i: (i, 0)),
            pl.BlockSpec((R // 128, 128), lambda i: (i, 0)),
            pl.BlockSpec((R, C), lambda i: (i, 0)),
        ],
        out_shape=[
            jax.ShapeDtypeStruct((N, C), jnp.float32),
            jax.ShapeDtypeStruct((N // 128, 128), jnp.int32),
            jax.ShapeDtypeStruct((N, C), jnp.float32),
        ],
    )(coefs, coefs, log_node_vt, v_pred, uniform)

    return (log_out, idx_out.reshape(N), oh_out)


# speedup vs baseline: 33.7012x; 1.0058x over previous
import functools

import jax
import jax.numpy as jnp
from jax import lax
from jax.experimental import pallas as pl
from jax.experimental.pallas import tpu as pltpu
from jax.experimental.pallas import tpu_sc as plsc

_ROWS = 2048
_L = 16


def _make_sc_gather(n_rows, n_graphs, n_t, rows_per_step):
    info = plsc.get_sparse_core_info()
    nw = info.num_cores * info.num_subcores
    b_per_w = n_rows // nw
    n_iters = b_per_w // _L
    chunks_per_step = rows_per_step // _L
    out_rows = 2 * n_rows // 128
    w_rows = out_rows // nw
    mesh = plsc.VectorSubcoreMesh(core_axis_name="c", subcore_axis_name="s")
    f32 = jnp.float32

    @functools.partial(
        pl.kernel,
        mesh=mesh,
        compiler_params=pltpu.CompilerParams(needs_layout_passes=False),
        out_type=jax.ShapeDtypeStruct((out_rows, 128), f32),
        scratch_types=[
            pltpu.VMEM((b_per_w,), jnp.int32),
            pltpu.VMEM((n_t,), f32),
            pltpu.VMEM((n_t,), f32),
            pltpu.VMEM((n_t,), f32),
            pltpu.VMEM((n_t,), f32),
            pltpu.VMEM((n_graphs,), jnp.int32),
            pltpu.VMEM((n_graphs,), f32),
            pltpu.VMEM((n_graphs,), f32),
            pltpu.VMEM((w_rows, 128), f32),
        ],
    )
    def gather_k(tbl_hbm, ts_hbm, batch_hbm, out_hbm,
                 idx_v, t0, t1, t2, t3, ts_v, gta, gtb, pv):
        wid = lax.axis_index("s") * info.num_cores + lax.axis_index("c")
        base = wid * b_per_w
        pltpu.sync_copy(tbl_hbm.at[0], t0)
        pltpu.sync_copy(tbl_hbm.at[1], t1)
        pltpu.sync_copy(tbl_hbm.at[2], t2)
        pltpu.sync_copy(tbl_hbm.at[3], t3)
        pltpu.sync_copy(ts_hbm, ts_v)
        pltpu.sync_copy(batch_hbm.at[pl.ds(base, b_per_w)], idx_v)

        inv_c = jnp.float32(1.0 / 128.0)

        def prep(i):
            t16 = ts_v[pl.ds(i * _L, _L)]
            tm1 = jnp.maximum(t16 - 1, 0)
            va = plsc.load_gather(t0, [tm1])
            vb = plsc.load_gather(t1, [tm1])
            vc = plsc.load_gather(t2, [t16])
            vd = plsc.load_gather(t3, [t16])
            ba = jnp.where(t16 == 0, jnp.float32(0.0),
                           jnp.exp(vb - va) * inv_c)
            rho = jnp.exp(vd - vc) * inv_c
            gta[pl.ds(i * _L, _L)] = ba
            gtb[pl.ds(i * _L, _L)] = rho

        pl.loop(0, n_graphs // _L)(prep)

        def body(i):
            g = idx_v[pl.ds(i * _L, _L)]
            j = i // chunks_per_step
            k0 = (i % chunks_per_step) * _L
            o = j * (2 * rows_per_step) + k0
            pv[(o // 128), pl.ds(o % 128, _L)] = plsc.load_gather(gta, [g])
            o = o + rows_per_step
            pv[(o // 128), pl.ds(o % 128, _L)] = plsc.load_gather(gtb, [g])

        pl.loop(0, n_iters)(body)
        pltpu.sync_copy(pv, out_hbm.at[pl.ds(wid * w_rows, w_rows), :])

    return gather_k


def _main_body(a_ref, b_ref, lv_ref, vp_ref, u_ref,
               log_out_ref, idx_out_ref, oh_out_ref):
    R, C = lv_ref.shape
    nsub = R // 128
    stack2 = jnp.concatenate([a_ref[:, :], b_ref[:, :]], axis=0)
    rio = jax.lax.broadcasted_iota(jnp.int32, (128, 128), 0)
    lio = jax.lax.broadcasted_iota(jnp.int32, (128, 128), 1)
    ident = (rio == lio).astype(jnp.float32)
    dnt = (((1,), (1,)), ((), ()))
    tc = jax.lax.dot_general(ident, stack2, dnt,
                             precision=jax.lax.Precision.HIGHEST,
                             preferred_element_type=jnp.float32)

    ba = jnp.concatenate([tc[:, j:j + 1] for j in range(nsub)], axis=0)
    rho = jnp.concatenate([tc[:, nsub + j:nsub + j + 1]
                           for j in range(nsub)], axis=0)

    e1 = jnp.exp(vp_ref[:, :])
    s = jnp.sum(e1, axis=1, keepdims=True)
    tau = ba * s
    x1 = e1 + tau
    x2 = jnp.exp(lv_ref[:, :]) + rho
    g = -jnp.log(u_ref[:, :] + 1e-30)
    w = (x1 * x2) / g

    wmax = jnp.max(w, axis=1, keepdims=True)
    cio = jax.lax.broadcasted_iota(jnp.int32, (R, C), 1).astype(jnp.float32)
    eq = w == wmax
    idx = jnp.min(jnp.where(eq, cio, jnp.float32(C)), axis=1, keepdims=True)
    oh_out_ref[:, :] = eq.astype(jnp.float32)
    log_out_ref[:, :] = jnp.where(eq, jnp.float32(0.0),
                                  jnp.log(jnp.float32(1e-30)))
    icols = jnp.concatenate(
        [idx[j * 128:(j + 1) * 128, :] for j in range(nsub)],
        axis=1)
    dnr = (((0,), (0,)), ((), ()))
    irows = jax.lax.dot_general(icols, ident, dnr,
                                preferred_element_type=jnp.float32)
    idx_out_ref[:, :] = irows.astype(jnp.int32)


@jax.jit
def kernel(log_node_vt, v_pred, timestep, batch, log_alphas, log_1_min_alphas,
           log_cumprod_alphas, log_1_min_cumprod_alphas, uniform):
    N, C = log_node_vt.shape
    B = timestep.shape[0]
    R = _ROWS

    T = log_alphas.shape[0]
    Tp = (T + 127) // 128 * 128
    tbl = jnp.stack([log_cumprod_alphas, log_1_min_cumprod_alphas,
                     log_alphas, log_1_min_alphas], axis=0)
    tbl = jnp.pad(tbl, ((0, 0), (0, Tp - T)))
    ts1 = timestep.astype(jnp.int32)
    batch1 = batch.astype(jnp.int32)

    coefs = _make_sc_gather(N, B, Tp, R)(tbl, ts1, batch1)

    rb = R // 128
    grid = (N // R,)
    log_out, idx_out, oh_out = pl.pallas_call(
        _main_body,
        grid=grid,
        in_specs=[
            pl.BlockSpec((rb, 128), lambda i: (2 * i + 0, 0)),
            pl.BlockSpec((rb, 128), lambda i: (2 * i + 1, 0)),
            pl.BlockSpec((R, C), lambda i: (i, 0)),
            pl.BlockSpec((R, C), lambda i: (i, 0)),
            pl.BlockSpec((R, C), lambda i: (i, 0)),
        ],
        out_specs=[
            pl.BlockSpec((R, C), lambda i: (i, 0)),
            pl.BlockSpec((R // 128, 128), lambda i: (i, 0)),
            pl.BlockSpec((R, C), lambda i: (i, 0)),
        ],
        out_shape=[
            jax.ShapeDtypeStruct((N, C), jnp.float32),
            jax.ShapeDtypeStruct((N // 128, 128), jnp.int32),
            jax.ShapeDtypeStruct((N, C), jnp.float32),
        ],
    )(coefs, coefs, log_node_vt, v_pred, uniform)

    return (log_out, idx_out.reshape(N), oh_out)


# speedup vs baseline: 37.3388x; 1.1079x over previous
import functools

import jax
import jax.numpy as jnp
from jax import lax
from jax.experimental import pallas as pl
from jax.experimental.pallas import tpu as pltpu
from jax.experimental.pallas import tpu_sc as plsc

_ROWS = 4096
_L = 16


def _make_sc_gather(n_rows, n_graphs, n_t, rows_per_step):
    info = plsc.get_sparse_core_info()
    nw = info.num_cores * info.num_subcores
    b_per_w = n_rows // nw
    n_iters = b_per_w // _L
    chunks_per_step = rows_per_step // _L
    out_rows = 2 * n_rows // 128
    w_rows = out_rows // nw
    mesh = plsc.VectorSubcoreMesh(core_axis_name="c", subcore_axis_name="s")
    f32 = jnp.float32

    @functools.partial(
        pl.kernel,
        mesh=mesh,
        compiler_params=pltpu.CompilerParams(needs_layout_passes=False),
        out_type=jax.ShapeDtypeStruct((out_rows, 128), f32),
        scratch_types=[
            pltpu.VMEM((b_per_w,), jnp.int32),
            pltpu.VMEM((n_t,), f32),
            pltpu.VMEM((n_t,), f32),
            pltpu.VMEM((n_t,), f32),
            pltpu.VMEM((n_t,), f32),
            pltpu.VMEM((n_graphs,), jnp.int32),
            pltpu.VMEM((n_graphs,), f32),
            pltpu.VMEM((n_graphs,), f32),
            pltpu.VMEM((w_rows, 128), f32),
        ],
    )
    def gather_k(tbl_hbm, ts_hbm, batch_hbm, out_hbm,
                 idx_v, t0, t1, t2, t3, ts_v, gta, gtb, pv):
        wid = lax.axis_index("s") * info.num_cores + lax.axis_index("c")
        base = wid * b_per_w
        pltpu.sync_copy(tbl_hbm.at[0], t0)
        pltpu.sync_copy(tbl_hbm.at[1], t1)
        pltpu.sync_copy(tbl_hbm.at[2], t2)
        pltpu.sync_copy(tbl_hbm.at[3], t3)
        pltpu.sync_copy(ts_hbm, ts_v)
        pltpu.sync_copy(batch_hbm.at[pl.ds(base, b_per_w)], idx_v)

        inv_c = jnp.float32(1.0 / 128.0)

        def prep(i):
            t16 = ts_v[pl.ds(i * _L, _L)]
            tm1 = jnp.maximum(t16 - 1, 0)
            va = plsc.load_gather(t0, [tm1])
            vb = plsc.load_gather(t1, [tm1])
            vc = plsc.load_gather(t2, [t16])
            vd = plsc.load_gather(t3, [t16])
            ba = jnp.where(t16 == 0, jnp.float32(0.0),
                           jnp.exp(vb - va) * inv_c)
            rho = jnp.exp(vd - vc) * inv_c
            gta[pl.ds(i * _L, _L)] = ba
            gtb[pl.ds(i * _L, _L)] = rho

        pl.loop(0, n_graphs // _L)(prep)

        def body(i):
            g = idx_v[pl.ds(i * _L, _L)]
            j = i // chunks_per_step
            k0 = (i % chunks_per_step) * _L
            o = j * (2 * rows_per_step) + k0
            pv[(o // 128), pl.ds(o % 128, _L)] = plsc.load_gather(gta, [g])
            o = o + rows_per_step
            pv[(o // 128), pl.ds(o % 128, _L)] = plsc.load_gather(gtb, [g])

        pl.loop(0, n_iters, unroll=4)(body)
        pltpu.sync_copy(pv, out_hbm.at[pl.ds(wid * w_rows, w_rows), :])

    return gather_k


def _main_body(a_ref, b_ref, lv_ref, vp_ref, u_ref,
               log_out_ref, idx_out_ref, oh_out_ref):
    R, C = lv_ref.shape
    nsub = R // 128
    stack2 = jnp.concatenate([a_ref[:, :], b_ref[:, :]], axis=0)
    rio = jax.lax.broadcasted_iota(jnp.int32, (128, 128), 0)
    lio = jax.lax.broadcasted_iota(jnp.int32, (128, 128), 1)
    ident = (rio == lio).astype(jnp.float32)
    dnt = (((1,), (1,)), ((), ()))
    tc = jax.lax.dot_general(ident, stack2, dnt,
                             precision=jax.lax.Precision.HIGHEST,
                             preferred_element_type=jnp.float32)

    ba = jnp.concatenate([tc[:, j:j + 1] for j in range(nsub)], axis=0)
    rho = jnp.concatenate([tc[:, nsub + j:nsub + j + 1]
                           for j in range(nsub)], axis=0)

    e1 = jnp.exp(vp_ref[:, :])
    s = jnp.sum(e1, axis=1, keepdims=True)
    tau = ba * s
    x1 = e1 + tau
    x2 = jnp.exp(lv_ref[:, :]) + rho
    g = -jnp.log(u_ref[:, :] + 1e-30)
    w = (x1 * x2) / g

    wmax = jnp.max(w, axis=1, keepdims=True)
    cio = jax.lax.broadcasted_iota(jnp.int32, (R, C), 1).astype(jnp.float32)
    eq = w == wmax
    idx = jnp.min(jnp.where(eq, cio, jnp.float32(C)), axis=1, keepdims=True)
    oh_out_ref[:, :] = eq.astype(jnp.float32)
    log_out_ref[:, :] = jnp.where(eq, jnp.float32(0.0),
                                  jnp.log(jnp.float32(1e-30)))
    icols = jnp.concatenate(
        [idx[j * 128:(j + 1) * 128, :] for j in range(nsub)],
        axis=1)
    dnr = (((0,), (0,)), ((), ()))
    irows = jax.lax.dot_general(icols, ident, dnr,
                                preferred_element_type=jnp.float32)
    idx_out_ref[:, :] = irows.astype(jnp.int32)


@jax.jit
def kernel(log_node_vt, v_pred, timestep, batch, log_alphas, log_1_min_alphas,
           log_cumprod_alphas, log_1_min_cumprod_alphas, uniform):
    N, C = log_node_vt.shape
    B = timestep.shape[0]
    R = _ROWS

    T = log_alphas.shape[0]
    Tp = (T + 127) // 128 * 128
    tbl = jnp.stack([log_cumprod_alphas, log_1_min_cumprod_alphas,
                     log_alphas, log_1_min_alphas], axis=0)
    tbl = jnp.pad(tbl, ((0, 0), (0, Tp - T)))
    ts1 = timestep.astype(jnp.int32)
    batch1 = batch.astype(jnp.int32)

    coefs = _make_sc_gather(N, B, Tp, R)(tbl, ts1, batch1)

    rb = R // 128
    grid = (N // R,)
    log_out, idx_out, oh_out = pl.pallas_call(
        _main_body,
        grid=grid,
        in_specs=[
            pl.BlockSpec((rb, 128), lambda i: (2 * i + 0, 0)),
            pl.BlockSpec((rb, 128), lambda i: (2 * i + 1, 0)),
            pl.BlockSpec((R, C), lambda i: (i, 0)),
            pl.BlockSpec((R, C), lambda i: (i, 0)),
            pl.BlockSpec((R, C), lambda i: (i, 0)),
        ],
        out_specs=[
            pl.BlockSpec((R, C), lambda i: (i, 0)),
            pl.BlockSpec((R // 128, 128), lambda i: (i, 0)),
            pl.BlockSpec((R, C), lambda i: (i, 0)),
        ],
        out_shape=[
            jax.ShapeDtypeStruct((N, C), jnp.float32),
            jax.ShapeDtypeStruct((N // 128, 128), jnp.int32),
            jax.ShapeDtypeStruct((N, C), jnp.float32),
        ],
    )(coefs, coefs, log_node_vt, v_pred, uniform)

    return (log_out, idx_out.reshape(N), oh_out)


# speedup vs baseline: 38.2502x; 1.0244x over previous
import functools

import jax
import jax.numpy as jnp
from jax import lax
from jax.experimental import pallas as pl
from jax.experimental.pallas import tpu as pltpu
from jax.experimental.pallas import tpu_sc as plsc

_ROWS = 8192
_L = 16


def _make_sc_gather(n_rows, n_graphs, n_t, rows_per_step):
    info = plsc.get_sparse_core_info()
    nw = info.num_cores * info.num_subcores
    b_per_w = n_rows // nw
    n_iters = b_per_w // _L
    chunks_per_step = rows_per_step // _L
    out_rows = 2 * n_rows // 128
    w_rows = out_rows // nw
    mesh = plsc.VectorSubcoreMesh(core_axis_name="c", subcore_axis_name="s")
    f32 = jnp.float32

    @functools.partial(
        pl.kernel,
        mesh=mesh,
        compiler_params=pltpu.CompilerParams(needs_layout_passes=False),
        out_type=jax.ShapeDtypeStruct((out_rows, 128), f32),
        scratch_types=[
            pltpu.VMEM((b_per_w,), jnp.int32),
            pltpu.VMEM((n_t,), f32),
            pltpu.VMEM((n_t,), f32),
            pltpu.VMEM((n_t,), f32),
            pltpu.VMEM((n_t,), f32),
            pltpu.VMEM((n_graphs,), jnp.int32),
            pltpu.VMEM((n_graphs,), f32),
            pltpu.VMEM((n_graphs,), f32),
            pltpu.VMEM((w_rows, 128), f32),
        ],
    )
    def gather_k(tbl_hbm, ts_hbm, batch_hbm, out_hbm,
                 idx_v, t0, t1, t2, t3, ts_v, gta, gtb, pv):
        wid = lax.axis_index("s") * info.num_cores + lax.axis_index("c")
        base = wid * b_per_w
        pltpu.sync_copy(tbl_hbm.at[0], t0)
        pltpu.sync_copy(tbl_hbm.at[1], t1)
        pltpu.sync_copy(tbl_hbm.at[2], t2)
        pltpu.sync_copy(tbl_hbm.at[3], t3)
        pltpu.sync_copy(ts_hbm, ts_v)
        pltpu.sync_copy(batch_hbm.at[pl.ds(base, b_per_w)], idx_v)

        inv_c = jnp.float32(1.0 / 128.0)

        def prep(i):
            t16 = ts_v[pl.ds(i * _L, _L)]
            tm1 = jnp.maximum(t16 - 1, 0)
            va = plsc.load_gather(t0, [tm1])
            vb = plsc.load_gather(t1, [tm1])
            vc = plsc.load_gather(t2, [t16])
            vd = plsc.load_gather(t3, [t16])
            ba = jnp.where(t16 == 0, jnp.float32(0.0),
                           jnp.exp(vb - va) * inv_c)
            rho = jnp.exp(vd - vc) * inv_c
            gta[pl.ds(i * _L, _L)] = ba
            gtb[pl.ds(i * _L, _L)] = rho

        pl.loop(0, n_graphs // _L)(prep)

        def body(i):
            g = idx_v[pl.ds(i * _L, _L)]
            j = i // chunks_per_step
            k0 = (i % chunks_per_step) * _L
            o = j * (2 * rows_per_step) + k0
            pv[(o // 128), pl.ds(o % 128, _L)] = plsc.load_gather(gta, [g])
            o = o + rows_per_step
            pv[(o // 128), pl.ds(o % 128, _L)] = plsc.load_gather(gtb, [g])

        pl.loop(0, n_iters, unroll=4)(body)
        pltpu.sync_copy(pv, out_hbm.at[pl.ds(wid * w_rows, w_rows), :])

    return gather_k


def _main_body(a_ref, b_ref, lv_ref, vp_ref, u_ref,
               log_out_ref, idx_out_ref, oh_out_ref):
    R, C = lv_ref.shape
    nsub = R // 128
    stack2 = jnp.concatenate([a_ref[:, :], b_ref[:, :]], axis=0)
    rio = jax.lax.broadcasted_iota(jnp.int32, (128, 128), 0)
    lio = jax.lax.broadcasted_iota(jnp.int32, (128, 128), 1)
    ident = (rio == lio).astype(jnp.float32)
    dnt = (((1,), (1,)), ((), ()))
    tc = jax.lax.dot_general(ident, stack2, dnt,
                             precision=jax.lax.Precision.HIGHEST,
                             preferred_element_type=jnp.float32)

    ba = jnp.concatenate([tc[:, j:j + 1] for j in range(nsub)], axis=0)
    rho = jnp.concatenate([tc[:, nsub + j:nsub + j + 1]
                           for j in range(nsub)], axis=0)

    e1 = jnp.exp(vp_ref[:, :])
    s = jnp.sum(e1, axis=1, keepdims=True)
    tau = ba * s
    x1 = e1 + tau
    x2 = jnp.exp(lv_ref[:, :]) + rho
    g = -jnp.log(u_ref[:, :] + 1e-30)
    w = (x1 * x2) / g

    wmax = jnp.max(w, axis=1, keepdims=True)
    cio = jax.lax.broadcasted_iota(jnp.int32, (R, C), 1).astype(jnp.float32)
    eq = w == wmax
    idx = jnp.min(jnp.where(eq, cio, jnp.float32(C)), axis=1, keepdims=True)
    oh_out_ref[:, :] = eq.astype(jnp.float32)
    log_out_ref[:, :] = jnp.where(eq, jnp.float32(0.0),
                                  jnp.log(jnp.float32(1e-30)))
    icols = jnp.concatenate(
        [idx[j * 128:(j + 1) * 128, :] for j in range(nsub)],
        axis=1)
    dnr = (((0,), (0,)), ((), ()))
    irows = jax.lax.dot_general(icols, ident, dnr,
                                preferred_element_type=jnp.float32)
    idx_out_ref[:, :] = irows.astype(jnp.int32)


@jax.jit
def kernel(log_node_vt, v_pred, timestep, batch, log_alphas, log_1_min_alphas,
           log_cumprod_alphas, log_1_min_cumprod_alphas, uniform):
    N, C = log_node_vt.shape
    B = timestep.shape[0]
    R = _ROWS

    T = log_alphas.shape[0]
    Tp = (T + 127) // 128 * 128
    tbl = jnp.stack([log_cumprod_alphas, log_1_min_cumprod_alphas,
                     log_alphas, log_1_min_alphas], axis=0)
    tbl = jnp.pad(tbl, ((0, 0), (0, Tp - T)))
    ts1 = timestep.astype(jnp.int32)
    batch1 = batch.astype(jnp.int32)

    coefs = _make_sc_gather(N, B, Tp, R)(tbl, ts1, batch1)

    rb = R // 128
    grid = (N // R,)
    log_out, idx_out, oh_out = pl.pallas_call(
        _main_body,
        grid=grid,
        in_specs=[
            pl.BlockSpec((rb, 128), lambda i: (2 * i + 0, 0)),
            pl.BlockSpec((rb, 128), lambda i: (2 * i + 1, 0)),
            pl.BlockSpec((R, C), lambda i: (i, 0)),
            pl.BlockSpec((R, C), lambda i: (i, 0)),
            pl.BlockSpec((R, C), lambda i: (i, 0)),
        ],
        out_specs=[
            pl.BlockSpec((R, C), lambda i: (i, 0)),
            pl.BlockSpec((R // 128, 128), lambda i: (i, 0)),
            pl.BlockSpec((R, C), lambda i: (i, 0)),
        ],
        out_shape=[
            jax.ShapeDtypeStruct((N, C), jnp.float32),
            jax.ShapeDtypeStruct((N // 128, 128), jnp.int32),
            jax.ShapeDtypeStruct((N, C), jnp.float32),
        ],
    )(coefs, coefs, log_node_vt, v_pred, uniform)

    return (log_out, idx_out.reshape(N), oh_out)


# speedup vs baseline: 38.2722x; 1.0006x over previous
import functools

import jax
import jax.numpy as jnp
from jax import lax
from jax.experimental import pallas as pl
from jax.experimental.pallas import tpu as pltpu
from jax.experimental.pallas import tpu_sc as plsc

_ROWS = 8192
_L = 16


def _make_sc_gather(n_rows, n_graphs, n_t, rows_per_step):
    info = plsc.get_sparse_core_info()
    nw = info.num_cores * info.num_subcores
    b_per_w = n_rows // nw
    n_iters = b_per_w // _L
    chunks_per_step = rows_per_step // _L
    out_rows = 2 * n_rows // 128
    w_rows = out_rows // nw
    mesh = plsc.VectorSubcoreMesh(core_axis_name="c", subcore_axis_name="s")
    f32 = jnp.float32

    @functools.partial(
        pl.kernel,
        mesh=mesh,
        compiler_params=pltpu.CompilerParams(needs_layout_passes=False),
        out_type=jax.ShapeDtypeStruct((out_rows, 128), f32),
        scratch_types=[
            pltpu.VMEM((b_per_w,), jnp.int32),
            pltpu.VMEM((n_t,), f32),
            pltpu.VMEM((n_t,), f32),
            pltpu.VMEM((n_t,), f32),
            pltpu.VMEM((n_t,), f32),
            pltpu.VMEM((n_graphs,), jnp.int32),
            pltpu.VMEM((n_graphs,), f32),
            pltpu.VMEM((n_graphs,), f32),
            pltpu.VMEM((w_rows, 128), f32),
        ],
    )
    def gather_k(tbl_hbm, ts_hbm, batch_hbm, out_hbm,
                 idx_v, t0, t1, t2, t3, ts_v, gta, gtb, pv):
        wid = lax.axis_index("s") * info.num_cores + lax.axis_index("c")
        base = wid * b_per_w
        pltpu.sync_copy(tbl_hbm.at[0], t0)
        pltpu.sync_copy(tbl_hbm.at[1], t1)
        pltpu.sync_copy(tbl_hbm.at[2], t2)
        pltpu.sync_copy(tbl_hbm.at[3], t3)
        pltpu.sync_copy(ts_hbm, ts_v)
        pltpu.sync_copy(batch_hbm.at[pl.ds(base, b_per_w)], idx_v)

        inv_c = jnp.float32(1.0 / 128.0)

        def prep(i):
            t16 = ts_v[pl.ds(i * _L, _L)]
            tm1 = jnp.maximum(t16 - 1, 0)
            va = plsc.load_gather(t0, [tm1])
            vb = plsc.load_gather(t1, [tm1])
            vc = plsc.load_gather(t2, [t16])
            vd = plsc.load_gather(t3, [t16])
            ba = jnp.where(t16 == 0, jnp.float32(0.0),
                           jnp.exp(vb - va) * inv_c)
            rho = jnp.exp(vd - vc) * inv_c
            gta[pl.ds(i * _L, _L)] = ba
            gtb[pl.ds(i * _L, _L)] = rho

        pl.loop(0, n_graphs // _L)(prep)

        def body(i):
            g = idx_v[pl.ds(i * _L, _L)]
            o = i * _L
            pv[(o // 128), pl.ds(o % 128, _L)] = plsc.load_gather(gta, [g])
            o = o + b_per_w
            pv[(o // 128), pl.ds(o % 128, _L)] = plsc.load_gather(gtb, [g])

        pl.loop(0, n_iters, unroll=4)(body)
        half = b_per_w // 128
        flat0 = (base // rows_per_step) * (2 * rows_per_step) \
            + (base % rows_per_step)
        r0 = pl.multiple_of(flat0 // 128, half)
        r1 = pl.multiple_of((flat0 + rows_per_step) // 128, half)
        pltpu.sync_copy(pv.at[pl.ds(0, half), :],
                        out_hbm.at[pl.ds(r0, half), :])
        pltpu.sync_copy(pv.at[pl.ds(half, half), :],
                        out_hbm.at[pl.ds(r1, half), :])

    return gather_k


def _main_body(a_ref, b_ref, lv_ref, vp_ref, u_ref,
               log_out_ref, idx_out_ref, oh_out_ref):
    R, C = lv_ref.shape
    nsub = R // 128
    stack2 = jnp.concatenate([a_ref[:, :], b_ref[:, :]], axis=0)
    rio = jax.lax.broadcasted_iota(jnp.int32, (128, 128), 0)
    lio = jax.lax.broadcasted_iota(jnp.int32, (128, 128), 1)
    ident = (rio == lio).astype(jnp.float32)
    dnt = (((1,), (1,)), ((), ()))
    tc = jax.lax.dot_general(ident, stack2, dnt,
                             precision=jax.lax.Precision.HIGHEST,
                             preferred_element_type=jnp.float32)

    ba = jnp.concatenate([tc[:, j:j + 1] for j in range(nsub)], axis=0)
    rho = jnp.concatenate([tc[:, nsub + j:nsub + j + 1]
                           for j in range(nsub)], axis=0)

    e1 = jnp.exp(vp_ref[:, :])
    s = jnp.sum(e1, axis=1, keepdims=True)
    tau = ba * s
    x1 = e1 + tau
    x2 = jnp.exp(lv_ref[:, :]) + rho
    g = -jnp.log(u_ref[:, :] + 1e-30)
    w = (x1 * x2) / g

    wmax = jnp.max(w, axis=1, keepdims=True)
    cio = jax.lax.broadcasted_iota(jnp.int32, (R, C), 1).astype(jnp.float32)
    eq = w == wmax
    idx = jnp.min(jnp.where(eq, cio, jnp.float32(C)), axis=1, keepdims=True)
    oh_out_ref[:, :] = eq.astype(jnp.float32)
    log_out_ref[:, :] = jnp.where(eq, jnp.float32(0.0),
                                  jnp.log(jnp.float32(1e-30)))
    icols = jnp.concatenate(
        [idx[j * 128:(j + 1) * 128, :] for j in range(nsub)],
        axis=1)
    dnr = (((0,), (0,)), ((), ()))
    irows = jax.lax.dot_general(icols, ident, dnr,
                                preferred_element_type=jnp.float32)
    idx_out_ref[:, :] = irows.astype(jnp.int32)


@jax.jit
def kernel(log_node_vt, v_pred, timestep, batch, log_alphas, log_1_min_alphas,
           log_cumprod_alphas, log_1_min_cumprod_alphas, uniform):
    N, C = log_node_vt.shape
    B = timestep.shape[0]
    R = _ROWS

    T = log_alphas.shape[0]
    Tp = (T + 127) // 128 * 128
    tbl = jnp.stack([log_cumprod_alphas, log_1_min_cumprod_alphas,
                     log_alphas, log_1_min_alphas], axis=0)
    tbl = jnp.pad(tbl, ((0, 0), (0, Tp - T)))
    ts1 = timestep.astype(jnp.int32)
    batch1 = batch.astype(jnp.int32)

    coefs = _make_sc_gather(N, B, Tp, R)(tbl, ts1, batch1)

    rb = R // 128
    grid = (N // R,)
    log_out, idx_out, oh_out = pl.pallas_call(
        _main_body,
        grid=grid,
        in_specs=[
            pl.BlockSpec((rb, 128), lambda i: (2 * i + 0, 0)),
            pl.BlockSpec((rb, 128), lambda i: (2 * i + 1, 0)),
            pl.BlockSpec((R, C), lambda i: (i, 0)),
            pl.BlockSpec((R, C), lambda i: (i, 0)),
            pl.BlockSpec((R, C), lambda i: (i, 0)),
        ],
        out_specs=[
            pl.BlockSpec((R, C), lambda i: (i, 0)),
            pl.BlockSpec((R // 128, 128), lambda i: (i, 0)),
            pl.BlockSpec((R, C), lambda i: (i, 0)),
        ],
        out_shape=[
            jax.ShapeDtypeStruct((N, C), jnp.float32),
            jax.ShapeDtypeStruct((N // 128, 128), jnp.int32),
            jax.ShapeDtypeStruct((N, C), jnp.float32),
        ],
    )(coefs, coefs, log_node_vt, v_pred, uniform)

    return (log_out, idx_out.reshape(N), oh_out)


# speedup vs baseline: 38.2855x; 1.0003x over previous
import functools

import jax
import jax.numpy as jnp
from jax import lax
from jax.experimental import pallas as pl
from jax.experimental.pallas import tpu as pltpu
from jax.experimental.pallas import tpu_sc as plsc

_ROWS = 8192
_L = 16


def _make_sc_gather(n_rows, n_graphs, n_t, rows_per_step):
    info = plsc.get_sparse_core_info()
    nw = info.num_cores * info.num_subcores
    b_per_w = n_rows // nw
    n_iters = b_per_w // _L
    assert rows_per_step % b_per_w == 0
    out_rows = 2 * n_rows // 128
    w_rows = out_rows // nw
    mesh = plsc.VectorSubcoreMesh(core_axis_name="c", subcore_axis_name="s")
    f32 = jnp.float32

    @functools.partial(
        pl.kernel,
        mesh=mesh,
        compiler_params=pltpu.CompilerParams(needs_layout_passes=False),
        out_type=jax.ShapeDtypeStruct((out_rows, 128), f32),
        scratch_types=[
            pltpu.VMEM((b_per_w,), jnp.int32),
            pltpu.VMEM((n_t,), f32),
            pltpu.VMEM((n_t,), f32),
            pltpu.VMEM((n_t,), f32),
            pltpu.VMEM((n_t,), f32),
            pltpu.VMEM((n_graphs,), jnp.int32),
            pltpu.VMEM((n_graphs,), f32),
            pltpu.VMEM((n_graphs,), f32),
            pltpu.VMEM((w_rows, 128), f32),
        ],
    )
    def gather_k(tbl_hbm, ts_hbm, batch_hbm, out_hbm,
                 idx_v, t0, t1, t2, t3, ts_v, gta, gtb, pv):
        wid = lax.axis_index("s") * info.num_cores + lax.axis_index("c")
        base = wid * b_per_w
        pltpu.sync_copy(tbl_hbm.at[0], t0)
        pltpu.sync_copy(tbl_hbm.at[1], t1)
        pltpu.sync_copy(tbl_hbm.at[2], t2)
        pltpu.sync_copy(tbl_hbm.at[3], t3)
        pltpu.sync_copy(ts_hbm, ts_v)
        pltpu.sync_copy(batch_hbm.at[pl.ds(base, b_per_w)], idx_v)

        inv_c = jnp.float32(1.0 / 128.0)

        def prep(i):
            t16 = ts_v[pl.ds(i * _L, _L)]
            tm1 = jnp.maximum(t16 - 1, 0)
            va = plsc.load_gather(t0, [tm1])
            vb = plsc.load_gather(t1, [tm1])
            vc = plsc.load_gather(t2, [t16])
            vd = plsc.load_gather(t3, [t16])
            ba = jnp.where(t16 == 0, jnp.float32(0.0),
                           jnp.exp(vb - va) * inv_c)
            rho = jnp.exp(vd - vc) * inv_c
            gta[pl.ds(i * _L, _L)] = ba
            gtb[pl.ds(i * _L, _L)] = rho

        pl.loop(0, n_graphs // _L)(prep)

        def body(i):
            g = idx_v[pl.ds(i * _L, _L)]
            o = i * _L
            pv[(o // 128), pl.ds(o % 128, _L)] = plsc.load_gather(gta, [g])
            o = o + b_per_w
            pv[(o // 128), pl.ds(o % 128, _L)] = plsc.load_gather(gtb, [g])

        pl.loop(0, n_iters, unroll=4)(body)
        half = b_per_w // 128
        flat0 = (base // rows_per_step) * (2 * rows_per_step) \
            + (base % rows_per_step)
        r0 = pl.multiple_of(flat0 // 128, half)
        r1 = pl.multiple_of((flat0 + rows_per_step) // 128, half)
        pltpu.sync_copy(pv.at[pl.ds(0, half), :],
                        out_hbm.at[pl.ds(r0, half), :])
        pltpu.sync_copy(pv.at[pl.ds(half, half), :],
                        out_hbm.at[pl.ds(r1, half), :])

    return gather_k


def _main_body(a_ref, b_ref, lv_ref, vp_ref, u_ref,
               log_out_ref, idx_out_ref, oh_out_ref):
    R, C = lv_ref.shape
    nsub = R // 128
    stack2 = jnp.concatenate([a_ref[:, :], b_ref[:, :]], axis=0)
    rio = jax.lax.broadcasted_iota(jnp.int32, (128, 128), 0)
    lio = jax.lax.broadcasted_iota(jnp.int32, (128, 128), 1)
    ident = (rio == lio).astype(jnp.float32)
    dnt = (((1,), (1,)), ((), ()))
    tc = jax.lax.dot_general(ident, stack2, dnt,
                             precision=jax.lax.Precision.HIGHEST,
                             preferred_element_type=jnp.float32)

    ba = jnp.concatenate([tc[:, j:j + 1] for j in range(nsub)], axis=0)
    rho = jnp.concatenate([tc[:, nsub + j:nsub + j + 1]
                           for j in range(nsub)], axis=0)

    e1 = jnp.exp(vp_ref[:, :])
    s = jnp.sum(e1, axis=1, keepdims=True)
    tau = ba * s
    x1 = e1 + tau
    x2 = jnp.exp(lv_ref[:, :]) + rho
    g = -jnp.log(u_ref[:, :] + 1e-30)
    w = (x1 * x2) / g

    wmax = jnp.max(w, axis=1, keepdims=True)
    cio = jax.lax.broadcasted_iota(jnp.int32, (R, C), 1).astype(jnp.float32)
    eq = w == wmax
    idx = jnp.min(jnp.where(eq, cio, jnp.float32(C)), axis=1, keepdims=True)
    oh_out_ref[:, :] = eq.astype(jnp.float32)
    log_out_ref[:, :] = jnp.where(eq, jnp.float32(0.0),
                                  jnp.log(jnp.float32(1e-30)))
    icols = jnp.concatenate(
        [idx[j * 128:(j + 1) * 128, :] for j in range(nsub)],
        axis=1)
    dnr = (((0,), (0,)), ((), ()))
    irows = jax.lax.dot_general(icols, ident, dnr,
                                preferred_element_type=jnp.float32)
    idx_out_ref[:, :] = irows.astype(jnp.int32)


@jax.jit
def kernel(log_node_vt, v_pred, timestep, batch, log_alphas, log_1_min_alphas,
           log_cumprod_alphas, log_1_min_cumprod_alphas, uniform):
    N, C = log_node_vt.shape
    B = timestep.shape[0]
    R = _ROWS

    T = log_alphas.shape[0]
    Tp = (T + 127) // 128 * 128
    tbl = jnp.stack([log_cumprod_alphas, log_1_min_cumprod_alphas,
                     log_alphas, log_1_min_alphas], axis=0)
    tbl = jnp.pad(tbl, ((0, 0), (0, Tp - T)))
    ts1 = timestep.astype(jnp.int32)
    batch1 = batch.astype(jnp.int32)

    coefs = _make_sc_gather(N, B, Tp, R)(tbl, ts1, batch1)

    rb = R // 128
    grid = (N // R,)
    log_out, idx_out, oh_out = pl.pallas_call(
        _main_body,
        grid=grid,
        in_specs=[
            pl.BlockSpec((rb, 128), lambda i: (2 * i + 0, 0)),
            pl.BlockSpec((rb, 128), lambda i: (2 * i + 1, 0)),
            pl.BlockSpec((R, C), lambda i: (i, 0)),
            pl.BlockSpec((R, C), lambda i: (i, 0)),
            pl.BlockSpec((R, C), lambda i: (i, 0)),
        ],
        out_specs=[
            pl.BlockSpec((R, C), lambda i: (i, 0)),
            pl.BlockSpec((R // 128, 128), lambda i: (i, 0)),
            pl.BlockSpec((R, C), lambda i: (i, 0)),
        ],
        out_shape=[
            jax.ShapeDtypeStruct((N, C), jnp.float32),
            jax.ShapeDtypeStruct((N // 128, 128), jnp.int32),
            jax.ShapeDtypeStruct((N, C), jnp.float32),
        ],
    )(coefs, coefs, log_node_vt, v_pred, uniform)

    return (log_out, idx_out.reshape(N), oh_out)
